# Initial kernel scaffold; baseline (speedup 1.0000x reference)
import jax
import jax.numpy as jnp
from jax.experimental import pallas as pl


def kernel(x, edge_index, batch, params):
    raise NotImplementedError("write your pallas kernel here")


# speedup vs baseline: 6.5967x; 6.5967x over previous
import functools

import jax
import jax.numpy as jnp
from jax import lax
from jax.experimental import pallas as pl
from jax.experimental.pallas import tpu as pltpu
from jax.experimental.pallas import tpu_sc as plsc

_N = 10000
_E = 320000
_D = 128
_G = 64
_EPS = 1e-5

_NC = 2
_NS = 16
_NW = _NC * _NS
_CH = 80
_EPW = _E // _NW
_RPW = _EPW // _CH
_NB = 5
_CPB = _RPW // _NB
_NPS = _N // _NS


def _make_sc_agg():
    mesh = plsc.VectorSubcoreMesh(
        core_axis_name="c", subcore_axis_name="s",
        num_cores=_NC, num_subcores=_NS)
    out_type = jax.ShapeDtypeStruct((_NC, _NS, _NPS, _D), jnp.float32)
    scratch = [
        pltpu.VMEM((_CPB, _CH), jnp.int32),
        pltpu.VMEM((_CPB, _CH), jnp.int32),
        pltpu.VMEM((_CH, _D), jnp.float32),
        pltpu.VMEM_SHARED((_N, _D), jnp.float32),
        pltpu.SemaphoreType.DMA,
    ]

    def body(h_hbm, src_hbm, dst_hbm, zd_hbm, *refs):
        (out_hbm, idx_s, idx_d, rows_v, agg_sh, sem) = refs
        c = lax.axis_index("c")
        s = lax.axis_index("s")
        w = c * _NS + s

        pltpu.sync_copy(zd_hbm.at[s], agg_sh.at[pl.ds(s * _NPS, _NPS)])

        plsc.subcore_barrier()

        def blk(b, _):
            pltpu.sync_copy(src_hbm.at[w, b], idx_s)
            pltpu.sync_copy(dst_hbm.at[w, b], idx_d)

            def step(j, _):
                pltpu.async_copy(h_hbm.at[idx_s.at[j]], rows_v, sem).wait()
                pltpu.sync_copy(rows_v, agg_sh.at[idx_d.at[j]], add=True)
                return 0
            lax.fori_loop(0, _CPB, step, 0)
            return 0
        lax.fori_loop(0, _NB, blk, 0)

        plsc.subcore_barrier()
        pltpu.sync_copy(agg_sh.at[pl.ds(s * _NPS, _NPS)], out_hbm.at[c, s])

    return pl.kernel(body, out_type=out_type, mesh=mesh,
                     scratch_types=scratch)


def _make_sc_cnt():
    mesh = plsc.VectorSubcoreMesh(
        core_axis_name="c", subcore_axis_name="s",
        num_cores=_NC, num_subcores=_NS)
    out_type = jax.ShapeDtypeStruct((_NC, _NS, _NPS, _D), jnp.float32)
    scratch = [
        pltpu.VMEM((_CPB, _CH), jnp.int32),
        pltpu.VMEM((_CH, _D), jnp.float32),
        pltpu.VMEM_SHARED((_N, _D), jnp.float32),
    ]

    def body(dst_hbm, zd_hbm, ones_hbm, *refs):
        (out_hbm, idx_d, ones_v, cnt_sh) = refs
        c = lax.axis_index("c")
        s = lax.axis_index("s")
        w = c * _NS + s

        pltpu.sync_copy(zd_hbm.at[s], cnt_sh.at[pl.ds(s * _NPS, _NPS)])
        pltpu.sync_copy(ones_hbm, ones_v)
        plsc.subcore_barrier()

        def blk(b, _):
            pltpu.sync_copy(dst_hbm.at[w, b], idx_d)

            def step(j, _):
                pltpu.sync_copy(ones_v, cnt_sh.at[idx_d.at[j]], add=True)
                return 0
            lax.fori_loop(0, _CPB, step, 0)
            return 0
        lax.fori_loop(0, _NB, blk, 0)

        plsc.subcore_barrier()
        pltpu.sync_copy(cnt_sh.at[pl.ds(s * _NPS, _NPS)], out_hbm.at[c, s])

    return pl.kernel(body, out_type=out_type, mesh=mesh,
                     scratch_types=scratch)


@functools.lru_cache(maxsize=None)
def _sc_agg_fn():
    return _make_sc_agg()


@functools.lru_cache(maxsize=None)
def _sc_cnt_fn():
    return _make_sc_cnt()


def _mlp_body(h_ref, p_ref, c_ref, pool_ref, w1_ref, b1_ref, g_ref, bt_ref,
              w2_ref, b2_ref, ho_ref, po_ref):
    cnt = c_ref[0, :, 0:1] + c_ref[1, :, 0:1]
    agg = (p_ref[0] + p_ref[1]) / jnp.maximum(cnt, 1.0)
    z = h_ref[...] + agg
    t = jnp.dot(z, w1_ref[...], preferred_element_type=jnp.float32) + b1_ref[...]
    mu = jnp.mean(t, axis=0, keepdims=True)
    d = t - mu
    var = jnp.mean(d * d, axis=0, keepdims=True)
    t = d * lax.rsqrt(var + _EPS) * g_ref[...] + bt_ref[...]
    t = jnp.maximum(t, 0.0)
    h = jnp.dot(t, w2_ref[...], preferred_element_type=jnp.float32) + b2_ref[...]
    ho_ref[...] = h
    po_ref[...] = pool_ref[...] + h


_tc_mlp = pl.pallas_call(
    _mlp_body,
    out_shape=(jax.ShapeDtypeStruct((_N, _D), jnp.float32),
               jax.ShapeDtypeStruct((_N, _D), jnp.float32)),
)


def _pool_body(pool_ref, b_ref, out_ref):
    oh = (b_ref[...] == lax.broadcasted_iota(jnp.int32, (1, _G), 1))
    oh = oh.astype(jnp.float32)
    cnts = jnp.sum(oh, axis=0, keepdims=True)
    ohn = oh / jnp.maximum(cnts, 1.0)
    out_ref[...] = lax.dot_general(
        ohn, pool_ref[...], (((0,), (0,)), ((), ())),
        preferred_element_type=jnp.float32)


_tc_pool = pl.pallas_call(
    _pool_body,
    out_shape=jax.ShapeDtypeStruct((_G, _D), jnp.float32),
)


def kernel(x, edge_index, batch, params):
    src2 = edge_index[0].reshape(_NW, _NB, _CPB, _CH)
    dst2 = edge_index[1].reshape(_NW, _NB, _CPB, _CH)
    b2 = batch.reshape(_N, 1)

    zd = jnp.zeros((_NS, _NPS, _D), jnp.float32)
    ones = jnp.ones((_CH, _D), jnp.float32)

    cntp = _sc_cnt_fn()(dst2, zd, ones).reshape(_NC, _N, _D)
    aggp = _sc_agg_fn()(x, src2, dst2, zd).reshape(_NC, _N, _D)
    pool = jnp.zeros((_N, _D), jnp.float32)
    h = x
    for l, (W1, b1, gm, bt, W2, b2_) in enumerate(params):
        if l > 0:
            aggp = _sc_agg_fn()(h, src2, dst2, zd).reshape(_NC, _N, _D)
        h, pool = _tc_mlp(h, aggp, cntp, pool, W1, b1.reshape(1, _D),
                          gm.reshape(1, _D), bt.reshape(1, _D), W2,
                          b2_.reshape(1, _D))
    gpool = _tc_pool(pool, b2)
    return (pool, gpool)


# speedup vs baseline: 9.5713x; 1.4509x over previous
import functools

import jax
import jax.numpy as jnp
from jax import lax
from jax.experimental import pallas as pl
from jax.experimental.pallas import tpu as pltpu
from jax.experimental.pallas import tpu_sc as plsc

_N = 10000
_E = 320000
_D = 128
_G = 64
_EPS = 1e-5

_NC = 2
_NS = 16
_NW = _NC * _NS
_CH = 80
_EPW = _E // _NW
_RPW = _EPW // _CH
_NB = 5
_CPB = _RPW // _NB
_NPS = _N // _NS


def _make_sc_agg():
    mesh = plsc.VectorSubcoreMesh(
        core_axis_name="c", subcore_axis_name="s",
        num_cores=_NC, num_subcores=_NS)
    out_type = jax.ShapeDtypeStruct((_NC, _NS, _NPS, _D), jnp.float32)
    scratch = [
        pltpu.VMEM((_CPB, _CH), jnp.int32),
        pltpu.VMEM((_CPB, _CH), jnp.int32),
        pltpu.VMEM((_CH, _D), jnp.float32),
        pltpu.VMEM((_CH, _D), jnp.float32),
        pltpu.VMEM_SHARED((_N, _D), jnp.float32),
        pltpu.SemaphoreType.DMA,
        pltpu.SemaphoreType.DMA,
    ]

    def body(h_hbm, src_hbm, dst_hbm, zd_hbm, *refs):
        (out_hbm, idx_s, idx_d, rows0, rows1, agg_sh, sem0, sem1) = refs
        c = lax.axis_index("c")
        s = lax.axis_index("s")
        w = c * _NS + s

        pltpu.sync_copy(zd_hbm.at[s], agg_sh.at[pl.ds(s * _NPS, _NPS)])

        plsc.subcore_barrier()

        def blk(b, _):
            pltpu.sync_copy(src_hbm.at[w, b], idx_s)
            pltpu.sync_copy(dst_hbm.at[w, b], idx_d)

            pltpu.async_copy(h_hbm.at[idx_s.at[0]], rows0, sem0)

            def step(i, _):
                j = 2 * i
                pltpu.async_copy(h_hbm.at[idx_s.at[j + 1]], rows1, sem1)
                pltpu.make_async_copy(h_hbm.at[idx_s.at[j]], rows0,
                                      sem0).wait()
                pltpu.sync_copy(rows0, agg_sh.at[idx_d.at[j]], add=True)
                pltpu.async_copy(h_hbm.at[idx_s.at[j + 2]], rows0, sem0)
                pltpu.make_async_copy(h_hbm.at[idx_s.at[j + 1]], rows1,
                                      sem1).wait()
                pltpu.sync_copy(rows1, agg_sh.at[idx_d.at[j + 1]],
                                add=True)
                return 0
            lax.fori_loop(0, (_CPB - 1) // 2, step, 0)

            pltpu.make_async_copy(h_hbm.at[idx_s.at[_CPB - 1]], rows0,
                                  sem0).wait()
            pltpu.sync_copy(rows0, agg_sh.at[idx_d.at[_CPB - 1]], add=True)
            return 0
        lax.fori_loop(0, _NB, blk, 0)

        plsc.subcore_barrier()
        pltpu.sync_copy(agg_sh.at[pl.ds(s * _NPS, _NPS)], out_hbm.at[c, s])

    return pl.kernel(body, out_type=out_type, mesh=mesh,
                     scratch_types=scratch)


def _make_sc_cnt():
    mesh = plsc.VectorSubcoreMesh(
        core_axis_name="c", subcore_axis_name="s",
        num_cores=_NC, num_subcores=_NS)
    out_type = jax.ShapeDtypeStruct((_NC, _NS, _NPS, _D), jnp.float32)
    scratch = [
        pltpu.VMEM((_CPB, _CH), jnp.int32),
        pltpu.VMEM((_CH, _D), jnp.float32),
        pltpu.VMEM_SHARED((_N, _D), jnp.float32),
    ]

    def body(dst_hbm, zd_hbm, ones_hbm, *refs):
        (out_hbm, idx_d, ones_v, cnt_sh) = refs
        c = lax.axis_index("c")
        s = lax.axis_index("s")
        w = c * _NS + s

        pltpu.sync_copy(zd_hbm.at[s], cnt_sh.at[pl.ds(s * _NPS, _NPS)])
        pltpu.sync_copy(ones_hbm, ones_v)
        plsc.subcore_barrier()

        def blk(b, _):
            pltpu.sync_copy(dst_hbm.at[w, b], idx_d)

            def step(j, _):
                pltpu.sync_copy(ones_v, cnt_sh.at[idx_d.at[j]], add=True)
                return 0
            lax.fori_loop(0, _CPB, step, 0)
            return 0
        lax.fori_loop(0, _NB, blk, 0)

        plsc.subcore_barrier()
        pltpu.sync_copy(cnt_sh.at[pl.ds(s * _NPS, _NPS)], out_hbm.at[c, s])

    return pl.kernel(body, out_type=out_type, mesh=mesh,
                     scratch_types=scratch)


@functools.lru_cache(maxsize=None)
def _sc_agg_fn():
    return _make_sc_agg()


@functools.lru_cache(maxsize=None)
def _sc_cnt_fn():
    return _make_sc_cnt()


def _mlp_body(h_ref, p_ref, c_ref, pool_ref, w1_ref, b1_ref, g_ref, bt_ref,
              w2_ref, b2_ref, ho_ref, po_ref):
    cnt = c_ref[0, :, 0:1] + c_ref[1, :, 0:1]
    agg = (p_ref[0] + p_ref[1]) / jnp.maximum(cnt, 1.0)
    z = h_ref[...] + agg
    t = jnp.dot(z, w1_ref[...], preferred_element_type=jnp.float32) + b1_ref[...]
    mu = jnp.mean(t, axis=0, keepdims=True)
    d = t - mu
    var = jnp.mean(d * d, axis=0, keepdims=True)
    t = d * lax.rsqrt(var + _EPS) * g_ref[...] + bt_ref[...]
    t = jnp.maximum(t, 0.0)
    h = jnp.dot(t, w2_ref[...], preferred_element_type=jnp.float32) + b2_ref[...]
    ho_ref[...] = h
    po_ref[...] = pool_ref[...] + h


_tc_mlp = pl.pallas_call(
    _mlp_body,
    out_shape=(jax.ShapeDtypeStruct((_N, _D), jnp.float32),
               jax.ShapeDtypeStruct((_N, _D), jnp.float32)),
)


def _pool_body(pool_ref, b_ref, out_ref):
    oh = (b_ref[...] == lax.broadcasted_iota(jnp.int32, (1, _G), 1))
    oh = oh.astype(jnp.float32)
    cnts = jnp.sum(oh, axis=0, keepdims=True)
    ohn = oh / jnp.maximum(cnts, 1.0)
    out_ref[...] = lax.dot_general(
        ohn, pool_ref[...], (((0,), (0,)), ((), ())),
        preferred_element_type=jnp.float32)


_tc_pool = pl.pallas_call(
    _pool_body,
    out_shape=jax.ShapeDtypeStruct((_G, _D), jnp.float32),
)


def kernel(x, edge_index, batch, params):
    src3 = edge_index[0].reshape(_NW, _NB, _CPB, _CH)
    dst3 = edge_index[1].reshape(_NW, _NB, _CPB, _CH)
    dst4 = dst3
    b2 = batch.reshape(_N, 1)

    zd = jnp.zeros((_NS, _NPS, _D), jnp.float32)
    ones = jnp.ones((_CH, _D), jnp.float32)

    cntp = _sc_cnt_fn()(dst4, zd, ones).reshape(_NC, _N, _D)
    aggp = _sc_agg_fn()(x, src3, dst3, zd).reshape(_NC, _N, _D)
    pool = jnp.zeros((_N, _D), jnp.float32)
    h = x
    for l, (W1, b1, gm, bt, W2, b2_) in enumerate(params):
        if l > 0:
            aggp = _sc_agg_fn()(h, src3, dst3, zd).reshape(_NC, _N, _D)
        h, pool = _tc_mlp(h, aggp, cntp, pool, W1, b1.reshape(1, _D),
                          gm.reshape(1, _D), bt.reshape(1, _D), W2,
                          b2_.reshape(1, _D))
    gpool = _tc_pool(pool, b2)
    return (pool, gpool)
